# baseline (speedup 1.0000x reference)
import functools

import jax
import jax.numpy as jnp
from jax import lax
from jax.experimental import pallas as pl
from jax.experimental.pallas import tpu as pltpu
from jax.experimental.pallas import tpu_sc as plsc

K = 128
FGBIT = 128
FGINC = 65536
KS = K + 1
KSCALE = K * (1.0 - 2.0 ** -9)
NC, NS, L = 2, 16, 16
NW = NC * NS
C = 21


def _codes_body(logits_ref, labels_ref, out_ref):
    x = logits_ref[...]
    lab = labels_ref[...]
    e = jnp.exp(x)
    p = e * (1.0 / jnp.sum(e, axis=1, keepdims=True))
    cls = lax.broadcasted_iota(jnp.int32, x.shape, 1)
    fg = lab[:, None, :, :] == cls
    fgf = fg.astype(jnp.float32)
    err = jnp.abs(fgf - p)
    b = (err * KSCALE).astype(jnp.int32)
    out_ref[...] = (b + jnp.where(fg, FGBIT, 0)).astype(jnp.int8)


def _hist_body(codes, out, buf0, buf1, hist, hist2, acc, sem0, sem1):
    wid = lax.axis_index("s") * NC + lax.axis_index("c")
    zeros16 = jnp.zeros((L,), jnp.int32)
    laneoff = lax.iota(jnp.int32, L) * KS
    bufs = (buf0, buf1)
    rows, cols = buf0.shape
    GRP = 2
    sems = (sem0, sem1)
    SPC = 2

    def zero_hist(i, _):
        hist[pl.ds(i * L, L)] = zeros16
        hist2[pl.ds(i * L, L)] = zeros16
        return 0
    lax.fori_loop(0, L * KS // L, zero_hist, 0)

    def strip_at(cc, j):
        k = wid + NW * j
        return codes.at[k // 16, cc, pl.ds((k % 16) * rows, rows), :]

    pltpu.async_copy(strip_at(0, 0), buf0, sem0)

    def class_body(c, _):
        for b in range(SPC):
            par = b % 2
            npar = (b + 1) % 2
            t1 = c * SPC + b + 1

            @pl.when(t1 < C * SPC)
            def _():
                pltpu.async_copy(strip_at(t1 // SPC, t1 % SPC),
                                 bufs[npar], sems[npar])

            pltpu.make_async_copy(strip_at(0, 0), bufs[par],
                                  sems[par]).wait()

            @plsc.parallel_loop(0, rows * cols // (64 * GRP), unroll=2)
            def vec_body(j):
                for g in range(GRP):
                    q = j * GRP + g
                    x64 = bufs[par][q // (cols // 64),
                                    pl.ds((q % (cols // 64)) * 64, 64)]
                    v = plsc.bitcast(x64, jnp.int32)
                    for sh in range(4):
                        w = (v >> (8 * sh)) & 0xFF
                        idx = (w & (FGBIT - 1)) + laneoff
                        val = jnp.where(w >= FGBIT, FGINC + 1, 1)
                        plsc.addupdate_scatter(hist if sh % 2 == 0 else hist2,
                                               [idx], val)

        def red_body(j, _):
            s = zeros16
            for l in range(L):
                o = l * KS + j * L
                s = s + hist[pl.ds(o, L)] + hist2[pl.ds(o, L)]
                hist[pl.ds(o, L)] = zeros16
                hist2[pl.ds(o, L)] = zeros16
            acc[pl.ds(c * K + j * L, L)] = (s & (FGINC - 1)).astype(jnp.float32)
            acc[pl.ds(C * K + c * K + j * L, L)] = (
                (s >> 16).astype(jnp.float32))
            return 0
        lax.fori_loop(0, K // L, red_body, 0)
        return 0
    lax.fori_loop(0, C, class_body, 0)
    pltpu.sync_copy(acc, out.at[wid])


def _reduce_body(h_ref, out_ref):
    h = h_ref[...]
    s = jnp.sum(h, axis=0)
    cnt = s[0]
    fgc = s[1]
    r = lax.broadcasted_iota(jnp.int32, (K, K), 0)
    q = lax.broadcasted_iota(jnp.int32, (K, K), 1)
    tri = (r <= q).astype(jnp.float32)
    pc = jnp.dot(cnt, tri, preferred_element_type=jnp.float32)
    pf = jnp.dot(fgc, tri, preferred_element_type=jnp.float32)
    ntot = pc[:, K - 1:K]
    g = pf[:, K - 1:K]
    p_suf = ntot - pc
    f_suf = g - pf
    b_suf = p_suf - f_suf
    jac = p_suf / jnp.maximum(g + b_suf, 1.0)
    sum_j = 1.0 + jnp.sum(jac, axis=1)
    loss_c = (sum_j - 0.5 * (1.0 + jac[:, K - 1])) / KSCALE
    present = g[:, 0] > 0.0
    total = jnp.sum(jnp.where(present, loss_c, 0.0))
    count = jnp.sum(present.astype(jnp.float32))
    val = jnp.where(count > 0, total / jnp.maximum(count, 1.0), 0.0)
    out_ref[...] = val * jnp.ones((1, 1), jnp.float32)


def _make_codes(logits, labels):
    B, Cc, H, W = logits.shape
    bh = 32
    return pl.pallas_call(
        _codes_body,
        grid=(B, H // bh),
        in_specs=[
            pl.BlockSpec((1, Cc, bh, W), lambda b, i: (b, 0, i, 0)),
            pl.BlockSpec((1, bh, W), lambda b, i: (b, i, 0)),
        ],
        out_specs=pl.BlockSpec((1, Cc, bh, W), lambda b, i: (b, 0, i, 0)),
        out_shape=jax.ShapeDtypeStruct((B, Cc, H, W), jnp.int8),
    )(logits, labels)


def _histogram(codes4d):
    nrows = codes4d.shape[2] // 16
    ncols = codes4d.shape[3]
    mesh = plsc.VectorSubcoreMesh(core_axis_name="c", subcore_axis_name="s")
    run = pl.kernel(
        _hist_body,
        out_type=jax.ShapeDtypeStruct((NW, 2 * C * K), jnp.float32),
        mesh=mesh,
        scratch_types=[
            pltpu.VMEM((nrows, ncols), jnp.int8),
            pltpu.VMEM((nrows, ncols), jnp.int8),
            pltpu.VMEM((L * KS,), jnp.int32),
            pltpu.VMEM((L * KS,), jnp.int32),
            pltpu.VMEM((2 * C * K,), jnp.float32),
            pltpu.SemaphoreType.DMA,
            pltpu.SemaphoreType.DMA,
        ],
        compiler_params=pltpu.CompilerParams(needs_layout_passes=False),
    )
    return run(codes4d)


def _reduce(hists):
    return pl.pallas_call(
        _reduce_body,
        out_shape=jax.ShapeDtypeStruct((1, 1), jnp.float32),
    )(hists)


def kernel(logits, labels):
    B, Cc, H, W = logits.shape
    codes = _make_codes(logits, labels)
    hists = _histogram(codes)
    out = _reduce(hists.reshape(NW, 2, C, K))
    return out[0, 0]


# speedup vs baseline: 1.0111x; 1.0111x over previous
import jax
import jax.numpy as jnp
from jax import lax
from jax.experimental import pallas as pl
from jax.experimental.pallas import tpu as pltpu
from jax.experimental.pallas import tpu_sc as plsc

K = 128
FGBIT = 128
FGINC = 65536
KS = K + 1
KSCALE = K * (1.0 - 2.0 ** -9)
NC, NS, L = 2, 16, 16
NW = NC * NS
C = 21


def _codes_body(logits_ref, labels_ref, out_ref):
    x = logits_ref[...]
    lab = labels_ref[...]
    e = jnp.exp(x)
    p = e * (1.0 / jnp.sum(e, axis=1, keepdims=True))
    cls = lax.broadcasted_iota(jnp.int32, x.shape, 1)
    fg = lab[:, None, :, :] == cls
    fgf = fg.astype(jnp.float32)
    err = jnp.abs(fgf - p)
    b = (err * KSCALE).astype(jnp.int32)
    out_ref[...] = (b + jnp.where(fg, FGBIT, 0)).astype(jnp.int8)


def _hist_body(codes, out, buf0, buf1, hist, acc, sem0, sem1):
    wid = lax.axis_index("s") * NC + lax.axis_index("c")
    zeros16 = jnp.zeros((L,), jnp.int32)
    laneoff = lax.iota(jnp.int32, L) * KS
    bufs = (buf0, buf1)
    rows, cols = buf0.shape
    GRP = 2
    sems = (sem0, sem1)
    SPC = 2

    def zero_hist(i, _):
        hist[pl.ds(i * L, L)] = zeros16
        return 0
    lax.fori_loop(0, L * KS // L, zero_hist, 0)

    def strip_at(cc, j):
        k = wid + NW * j
        return codes.at[k // 16, cc, pl.ds((k % 16) * rows, rows), :]

    pltpu.async_copy(strip_at(0, 0), buf0, sem0)

    def class_body(c, _):
        for b in range(SPC):
            par = b % 2
            npar = (b + 1) % 2
            t1 = c * SPC + b + 1

            @pl.when(t1 < C * SPC)
            def _():
                pltpu.async_copy(strip_at(t1 // SPC, t1 % SPC),
                                 bufs[npar], sems[npar])

            pltpu.make_async_copy(strip_at(0, 0), bufs[par],
                                  sems[par]).wait()

            @plsc.parallel_loop(0, rows * cols // (64 * GRP), unroll=2)
            def vec_body(j):
                for g in range(GRP):
                    q = j * GRP + g
                    x64 = bufs[par][q // (cols // 64),
                                    pl.ds((q % (cols // 64)) * 64, 64)]
                    v = plsc.bitcast(x64, jnp.int32)
                    for sh in range(4):
                        w = (v >> (8 * sh)) & 0xFF
                        idx = (w & (FGBIT - 1)) + laneoff
                        val = jnp.where(w >= FGBIT, FGINC + 1, 1)
                        plsc.addupdate_scatter(hist, [idx], val)

        def red_body(j, _):
            s = zeros16
            for l in range(L):
                o = l * KS + j * L
                s = s + hist[pl.ds(o, L)]
                hist[pl.ds(o, L)] = zeros16
            acc[pl.ds(c * K + j * L, L)] = (s & (FGINC - 1)).astype(jnp.float32)
            acc[pl.ds(C * K + c * K + j * L, L)] = (
                (s >> 16).astype(jnp.float32))
            return 0
        lax.fori_loop(0, K // L, red_body, 0)
        return 0
    lax.fori_loop(0, C, class_body, 0)
    pltpu.sync_copy(acc, out.at[wid])


def _reduce_body(h_ref, out_ref):
    h = h_ref[...]
    s = jnp.sum(h, axis=0)
    cnt = s[0]
    fgc = s[1]
    r = lax.broadcasted_iota(jnp.int32, (K, K), 0)
    q = lax.broadcasted_iota(jnp.int32, (K, K), 1)
    tri = (r <= q).astype(jnp.float32)
    pc = jnp.dot(cnt, tri, preferred_element_type=jnp.float32)
    pf = jnp.dot(fgc, tri, preferred_element_type=jnp.float32)
    ntot = pc[:, K - 1:K]
    g = pf[:, K - 1:K]
    p_suf = ntot - pc
    f_suf = g - pf
    b_suf = p_suf - f_suf
    jac = p_suf / jnp.maximum(g + b_suf, 1.0)
    sum_j = 1.0 + jnp.sum(jac, axis=1)
    loss_c = (sum_j - 0.5 * (1.0 + jac[:, K - 1])) / KSCALE
    present = g[:, 0] > 0.0
    total = jnp.sum(jnp.where(present, loss_c, 0.0))
    count = jnp.sum(present.astype(jnp.float32))
    val = jnp.where(count > 0, total / jnp.maximum(count, 1.0), 0.0)
    out_ref[...] = val * jnp.ones((1, 1), jnp.float32)


def _make_codes(logits, labels):
    B, Cc, H, W = logits.shape
    bh = 32
    return pl.pallas_call(
        _codes_body,
        grid=(B, H // bh),
        in_specs=[
            pl.BlockSpec((1, Cc, bh, W), lambda b, i: (b, 0, i, 0)),
            pl.BlockSpec((1, bh, W), lambda b, i: (b, i, 0)),
        ],
        out_specs=pl.BlockSpec((1, Cc, bh, W), lambda b, i: (b, 0, i, 0)),
        out_shape=jax.ShapeDtypeStruct((B, Cc, H, W), jnp.int8),
    )(logits, labels)


def _histogram(codes4d):
    nrows = codes4d.shape[2] // 16
    ncols = codes4d.shape[3]
    mesh = plsc.VectorSubcoreMesh(core_axis_name="c", subcore_axis_name="s")
    run = pl.kernel(
        _hist_body,
        out_type=jax.ShapeDtypeStruct((NW, 2 * C * K), jnp.float32),
        mesh=mesh,
        scratch_types=[
            pltpu.VMEM((nrows, ncols), jnp.int8),
            pltpu.VMEM((nrows, ncols), jnp.int8),
            pltpu.VMEM((L * KS,), jnp.int32),
            pltpu.VMEM((2 * C * K,), jnp.float32),
            pltpu.SemaphoreType.DMA,
            pltpu.SemaphoreType.DMA,
        ],
        compiler_params=pltpu.CompilerParams(needs_layout_passes=False),
    )
    return run(codes4d)


def _reduce(hists):
    return pl.pallas_call(
        _reduce_body,
        out_shape=jax.ShapeDtypeStruct((1, 1), jnp.float32),
    )(hists)


def kernel(logits, labels):
    B, Cc, H, W = logits.shape
    codes = _make_codes(logits, labels)
    hists = _histogram(codes)
    out = _reduce(hists.reshape(NW, 2, C, K))
    return out[0, 0]
